# Initial kernel scaffold; baseline (speedup 1.0000x reference)
import jax
import jax.numpy as jnp
from jax.experimental import pallas as pl


def kernel(seq1, seq2, seq3, seq4, adj, aug_adj1edge, aug_adj2edge, sparse, msk, samp_bias1, samp_bias2, lbl, sample, W0, b0, gamma0, beta0):
    raise NotImplementedError("write your pallas kernel here")


# speedup vs baseline: 1.7333x; 1.7333x over previous
import functools

import jax
import jax.numpy as jnp
from jax import lax
from jax.experimental import pallas as pl
from jax.experimental.pallas import tpu as pltpu
import jax.experimental.pallas.tpu_sc as plsc

N = 10000
F = 128
T = 10
NP = 10240
NW = 32
CPW = NP // NW
K = 8
NCH = CPW // K
BI = 400


def _gcn_body(x_ref, w_ref, b_ref, adj_ref, h_ref, xw_scr):
    @pl.when(pl.program_id(0) == 0)
    def _():
        xw_scr[...] = jnp.dot(x_ref[...], w_ref[...],
                              preferred_element_type=jnp.float32)

    z = jnp.dot(adj_ref[...], xw_scr[...],
                preferred_element_type=jnp.float32) + b_ref[...]
    h_ref[...] = jnp.where(z > 0, z, jnp.exp(jnp.minimum(z, 0.0)) - 1.0)


def _gcn(x, w, b, adj):
    return pl.pallas_call(
        _gcn_body,
        grid=(N // BI,),
        in_specs=[
            pl.BlockSpec((N, F), lambda i: (0, 0)),
            pl.BlockSpec((F, F), lambda i: (0, 0)),
            pl.BlockSpec((1, F), lambda i: (0, 0)),
            pl.BlockSpec((BI, N), lambda i: (i, 0)),
        ],
        out_specs=pl.BlockSpec((BI, F), lambda i: (i, 0)),
        out_shape=jax.ShapeDtypeStruct((N, F), jnp.float32),
        scratch_shapes=[pltpu.VMEM((N, F), jnp.float32)],
    )(x, w, b, adj)


def _bn_body(h_ref, gam_ref, bet_ref, out_ref):
    h = h_ref[...]
    mean = jnp.mean(h, axis=0, keepdims=True)
    c = h - mean
    var = jnp.mean(c * c, axis=0, keepdims=True)
    y = c * lax.rsqrt(var + 1e-5) * gam_ref[...] + bet_ref[...]
    rn = jnp.sqrt(jnp.sum(y * y, axis=1, keepdims=True))
    g = y / jnp.maximum(rn, 1e-8)
    out_ref[0:N, :] = g
    out_ref[N:NP, :] = jnp.zeros((NP - N, F), jnp.float32)


def _bn_normalize(h, gam, bet):
    return pl.pallas_call(
        _bn_body,
        out_shape=jax.ShapeDtypeStruct((NP, F), jnp.float32),
    )(h, gam, bet)


def _sc_sims_body(g_hbm, idx_hbm, out_hbm, qbuf, tbuf, idxbuf, simbuf, sem):
    wid = lax.axis_index("s") * 2 + lax.axis_index("c")
    base = wid * CPW
    lane = lax.iota(jnp.int32, 16)

    def chunk_body(ch, carry):
        bi = base + ch * K
        pltpu.sync_copy(idx_hbm.at[pl.ds(bi * T, K * T)], idxbuf)
        pltpu.sync_copy(g_hbm.at[pl.ds(bi, K)], qbuf)
        pltpu.async_copy(g_hbm.at[idxbuf], tbuf, sem).wait()

        def i_body(i, c2):
            qs = [qbuf[i, 16 * c:16 * (c + 1)] for c in range(8)]
            sim = jnp.zeros((16,), jnp.float32)
            for t in range(T):
                r = i * T + t
                acc = qs[0] * tbuf[r, 0:16]
                for c in range(1, 8):
                    acc = acc + qs[c] * tbuf[r, 16 * c:16 * (c + 1)]
                sim = jnp.where(lane == t, jnp.sum(acc), sim)
            simbuf[i, :] = sim
            return c2

        lax.fori_loop(0, K, i_body, 0)
        pltpu.sync_copy(simbuf, out_hbm.at[pl.ds(bi, K)])
        return carry

    lax.fori_loop(0, NCH, chunk_body, 0)


@functools.cache
def _sc_sims():
    return pl.kernel(
        _sc_sims_body,
        out_type=jax.ShapeDtypeStruct((NP, 16), jnp.float32),
        mesh=plsc.VectorSubcoreMesh(core_axis_name="c", subcore_axis_name="s"),
        compiler_params=pltpu.CompilerParams(needs_layout_passes=False),
        scratch_types=[
            pltpu.VMEM((K, F), jnp.float32),
            pltpu.VMEM((K * T, F), jnp.float32),
            pltpu.VMEM((K * T,), jnp.int32),
            pltpu.VMEM((K, 16), jnp.float32),
            pltpu.SemaphoreType.DMA,
        ],
    )


def _loss_body(sim_ref, out_ref):
    s = sim_ref[...]
    lane = lax.broadcasted_iota(jnp.int32, (NP, 16), 1)
    e = jnp.where((lane >= 1) & (lane < T), jnp.exp(s), 0.0)
    den = jnp.sum(e, axis=1, keepdims=True)
    li = jnp.log(den) - s[:, 0:1]
    row = lax.broadcasted_iota(jnp.int32, (NP, 1), 0)
    li = jnp.where(row < N, li, 0.0)
    out_ref[...] = (jnp.sum(li) / N).reshape(1, 1)


def _loss(sims):
    return pl.pallas_call(
        _loss_body,
        out_shape=jax.ShapeDtypeStruct((1, 1), jnp.float32),
    )(sims)


def kernel(seq1, seq2, seq3, seq4, adj, aug_adj1edge, aug_adj2edge, sparse,
           msk, samp_bias1, samp_bias2, lbl, sample, W0, b0, gamma0, beta0):
    x = seq1[0]
    h = _gcn(x, W0, b0.reshape(1, F), adj)
    g = _bn_normalize(h, gamma0.reshape(1, F), beta0.reshape(1, F))
    idx_flat = jnp.concatenate([
        sample.astype(jnp.int32).reshape(-1),
        jnp.zeros((NP - N) * T, jnp.int32),
    ])
    sims = _sc_sims()(g, idx_flat)
    return _loss(sims)[0, 0]


# speedup vs baseline: 2.1688x; 1.2512x over previous
import functools

import jax
import jax.numpy as jnp
from jax import lax
from jax.experimental import pallas as pl
from jax.experimental.pallas import tpu as pltpu
import jax.experimental.pallas.tpu_sc as plsc

N = 10000
F = 128
T = 10
NP = 10240
NW = 32
CPW = NP // NW
K = 8
NCH = CPW // K
BI = 400


def _gcn_body(x_ref, w_ref, b_ref, adj_ref, h_ref, xw_scr):
    @pl.when(pl.program_id(0) == 0)
    def _():
        xw_scr[...] = jnp.dot(x_ref[...], w_ref[...],
                              preferred_element_type=jnp.float32)

    z = jnp.dot(adj_ref[...], xw_scr[...],
                preferred_element_type=jnp.float32) + b_ref[...]
    h_ref[...] = jnp.where(z > 0, z, jnp.exp(jnp.minimum(z, 0.0)) - 1.0)


def _gcn(x, w, b, adj):
    return pl.pallas_call(
        _gcn_body,
        grid=(N // BI,),
        in_specs=[
            pl.BlockSpec((N, F), lambda i: (0, 0)),
            pl.BlockSpec((F, F), lambda i: (0, 0)),
            pl.BlockSpec((1, F), lambda i: (0, 0)),
            pl.BlockSpec((BI, N), lambda i: (i, 0)),
        ],
        out_specs=pl.BlockSpec((BI, F), lambda i: (i, 0)),
        out_shape=jax.ShapeDtypeStruct((N, F), jnp.float32),
        scratch_shapes=[pltpu.VMEM((N, F), jnp.float32)],
    )(x, w, b, adj)


def _bn_body(h_ref, gam_ref, bet_ref, out_ref):
    h = h_ref[...]
    mean = jnp.mean(h, axis=0, keepdims=True)
    c = h - mean
    var = jnp.mean(c * c, axis=0, keepdims=True)
    y = c * lax.rsqrt(var + 1e-5) * gam_ref[...] + bet_ref[...]
    rn = jnp.sqrt(jnp.sum(y * y, axis=1, keepdims=True))
    g = y / jnp.maximum(rn, 1e-8)
    out_ref[0:N, :] = g
    out_ref[N:NP, :] = jnp.zeros((NP - N, F), jnp.float32)


def _bn_normalize(h, gam, bet):
    return pl.pallas_call(
        _bn_body,
        out_shape=jax.ShapeDtypeStruct((NP, F), jnp.float32),
    )(h, gam, bet)


def _sc_sims_body(g_hbm, idx2_hbm, out_hbm, qall, idxall, tbufA, tbufB,
                  simall, semA, semB):
    wid = lax.axis_index("s") * 2 + lax.axis_index("c")
    base = wid * CPW
    lane = lax.iota(jnp.int32, 16)

    pltpu.sync_copy(idx2_hbm.at[pl.ds(wid * NCH, NCH)], idxall)
    pltpu.sync_copy(g_hbm.at[pl.ds(base, CPW)], qall)
    pltpu.async_copy(g_hbm.at[idxall.at[0]], tbufA, semA)
    pltpu.async_copy(g_hbm.at[idxall.at[1]], tbufB, semB)

    def compute_chunk(ch, tbuf):
        def i_body(i, c2):
            ii = ch * K + i
            qs = [qall[ii, 16 * c:16 * (c + 1)] for c in range(8)]
            sim = jnp.zeros((16,), jnp.float32)
            for t in range(T):
                r = i * T + t
                acc = qs[0] * tbuf[r, 0:16]
                for c in range(1, 8):
                    acc = acc + qs[c] * tbuf[r, 16 * c:16 * (c + 1)]
                sim = jnp.where(lane == t, jnp.sum(acc), sim)
            simall[ii, :] = sim
            return c2

        lax.fori_loop(0, K, i_body, 0)

    def pair_body(j, carry):
        ch = 2 * j
        pltpu.make_async_copy(g_hbm.at[idxall.at[0]], tbufA, semA).wait()
        compute_chunk(ch, tbufA)

        @pl.when(ch + 2 < NCH)
        def _():
            pltpu.async_copy(g_hbm.at[idxall.at[ch + 2]], tbufA, semA)

        pltpu.make_async_copy(g_hbm.at[idxall.at[0]], tbufB, semB).wait()
        compute_chunk(ch + 1, tbufB)

        @pl.when(ch + 3 < NCH)
        def _():
            pltpu.async_copy(g_hbm.at[idxall.at[ch + 3]], tbufB, semB)

        return carry

    lax.fori_loop(0, NCH // 2, pair_body, 0)
    pltpu.sync_copy(simall, out_hbm.at[pl.ds(base, CPW)])


@functools.cache
def _sc_sims():
    return pl.kernel(
        _sc_sims_body,
        out_type=jax.ShapeDtypeStruct((NP, 16), jnp.float32),
        mesh=plsc.VectorSubcoreMesh(core_axis_name="c", subcore_axis_name="s"),
        compiler_params=pltpu.CompilerParams(needs_layout_passes=False),
        scratch_types=[
            pltpu.VMEM((CPW, F), jnp.float32),
            pltpu.VMEM((NCH, K * T), jnp.int32),
            pltpu.VMEM((K * T, F), jnp.float32),
            pltpu.VMEM((K * T, F), jnp.float32),
            pltpu.VMEM((CPW, 16), jnp.float32),
            pltpu.SemaphoreType.DMA,
            pltpu.SemaphoreType.DMA,
        ],
    )


def _loss_body(sim_ref, out_ref):
    s = sim_ref[...]
    lane = lax.broadcasted_iota(jnp.int32, (NP, 16), 1)
    e = jnp.where((lane >= 1) & (lane < T), jnp.exp(s), 0.0)
    den = jnp.sum(e, axis=1, keepdims=True)
    li = jnp.log(den) - s[:, 0:1]
    row = lax.broadcasted_iota(jnp.int32, (NP, 1), 0)
    li = jnp.where(row < N, li, 0.0)
    out_ref[...] = (jnp.sum(li) / N).reshape(1, 1)


def _loss(sims):
    return pl.pallas_call(
        _loss_body,
        out_shape=jax.ShapeDtypeStruct((1, 1), jnp.float32),
    )(sims)


def kernel(seq1, seq2, seq3, seq4, adj, aug_adj1edge, aug_adj2edge, sparse,
           msk, samp_bias1, samp_bias2, lbl, sample, W0, b0, gamma0, beta0):
    x = seq1[0]
    h = _gcn(x, W0, b0.reshape(1, F), adj)
    g = _bn_normalize(h, gamma0.reshape(1, F), beta0.reshape(1, F))
    idx2 = jnp.concatenate([
        sample.astype(jnp.int32).reshape(-1),
        jnp.zeros((NP - N) * T, jnp.int32),
    ]).reshape(NP * T // (K * T), K * T)
    sims = _sc_sims()(g, idx2)
    return _loss(sims)[0, 0]
